# baseline (speedup 1.0000x reference)
import functools

import jax
import jax.numpy as jnp
from jax import lax
from jax.experimental import pallas as pl
from jax.experimental.pallas import tpu as pltpu
from jax.experimental.pallas import tpu_sc as plsc

N, P2, TOPK, W2, CKV = 8, 49, 8, 16, 768
SEC = P2 * TOPK
ROWS = N * SEC
NC, NS = 2, 16
HW = W2 // 2
NGRP = 27
SPS = 4

_mesh = plsc.VectorSubcoreMesh(core_axis_name="c", subcore_axis_name="s")


@functools.partial(
    pl.kernel,
    mesh=_mesh,
    out_type=jax.ShapeDtypeStruct((ROWS, W2, CKV), jnp.float32),
    scratch_types=[
        pltpu.VMEM((16 * NGRP,), jnp.int32),
        pltpu.VMEM_SHARED((SPS, P2, HW, CKV), jnp.float32),
        pltpu.SemaphoreType.DMA,
        pltpu.SemaphoreType.DMA,
    ],
)
def _gather_kernel(idx_hbm, tbl_hbm, out_hbm, idx_v, spm, gsem, wsem):
    wid = lax.axis_index("c") * NS + lax.axis_index("s")
    sec = wid // 4
    q = lax.rem(wid, 4)
    slot = lax.rem(sec, SPS)
    sec_row = sec * SEC
    tbl_base = sec * P2
    g0 = jnp.where(q == 0, 0, 6 * q + 1)
    n_g = jnp.where(q == 0, 7, 6)

    pltpu.sync_copy(idx_hbm.at[pl.ds(sec_row, SEC)], idx_v.at[pl.ds(0, SEC)])
    lanes = lax.iota(jnp.int32, 16)
    neg1 = jnp.full((16,), -1, jnp.int32)
    tail = idx_v[pl.ds(384, 16)]
    idx_v[pl.ds(384, 16)] = jnp.where(lanes < 8, tail, neg1)
    idx_v[pl.ds(400, 16)] = neg1
    idx_v[pl.ds(416, 16)] = neg1

    for h in range(2):
        @pl.when(q == 0)
        def _():
            pltpu.async_copy(
                tbl_hbm.at[pl.ds(tbl_base, 13), pl.ds(h * HW, HW)],
                spm.at[slot, pl.ds(0, 13)], gsem)
            pltpu.make_async_copy(
                tbl_hbm.at[pl.ds(0, 13), pl.ds(h * HW, HW)],
                spm.at[slot, pl.ds(0, 13)], gsem).wait()

        @pl.when(q > 0)
        def _():
            start = 12 * q + 1
            pltpu.async_copy(
                tbl_hbm.at[pl.ds(tbl_base + start, 12), pl.ds(h * HW, HW)],
                spm.at[slot, pl.ds(start, 12)], gsem)
            pltpu.make_async_copy(
                tbl_hbm.at[pl.ds(0, 12), pl.ds(h * HW, HW)],
                spm.at[slot, pl.ds(0, 12)], gsem).wait()

        plsc.subcore_barrier()

        def fan_group(g, nw):
            grp = g0 + g
            v = idx_v[pl.ds(16 * grp, 16)]
            for j in range(16):
                t = v[j]
                p = grp * 16 + j

                @pl.when(t >= 0)
                def _():
                    pltpu.async_copy(
                        spm.at[slot, t],
                        out_hbm.at[sec_row + p, pl.ds(h * HW, HW)], wsem)

                nw = nw + jnp.where(t >= 0, 1, 0)
            return nw

        nwrites = lax.fori_loop(0, n_g, fan_group, jnp.int32(0))

        def drain1(j, c):
            pltpu.make_async_copy(
                spm.at[slot, 0],
                out_hbm.at[0, pl.ds(h * HW, HW)], wsem).wait()
            return c

        lax.fori_loop(0, nwrites, drain1, 0)
        plsc.subcore_barrier()


def kernel(r_idx, r_weight, kv):
    del r_weight
    idx = r_idx.reshape(ROWS).astype(jnp.int32)
    tbl = kv.reshape(N * P2, W2, CKV)
    out = _gather_kernel(idx, tbl)
    return out.reshape(N, P2, TOPK, W2, CKV)


# speedup vs baseline: 1.0932x; 1.0932x over previous
import functools

import jax
import jax.numpy as jnp
from jax import lax
from jax.experimental import pallas as pl
from jax.experimental.pallas import tpu as pltpu
from jax.experimental.pallas import tpu_sc as plsc

N, P2, TOPK, W2, CKV = 8, 49, 8, 16, 768
SEC = P2 * TOPK
ROWS = N * SEC
NC, NS = 2, 16
CW = CKV // 3
NGRP = 27
SPS = 4

_mesh = plsc.VectorSubcoreMesh(core_axis_name="c", subcore_axis_name="s")


@functools.partial(
    pl.kernel,
    mesh=_mesh,
    out_type=jax.ShapeDtypeStruct((ROWS, W2, CKV), jnp.float32),
    scratch_types=[
        pltpu.VMEM((16 * NGRP,), jnp.int32),
        pltpu.VMEM_SHARED((SPS, P2, W2, CW), jnp.float32),
        pltpu.VMEM_SHARED((SPS, P2, W2, CW), jnp.float32),
        pltpu.SemaphoreType.DMA,
        pltpu.SemaphoreType.DMA,
    ],
)
def _gather_kernel(idx_hbm, tbl_hbm, out_hbm, idx_v, spm_a, spm_b, gsem,
                   wsem):
    wid = lax.axis_index("c") * NS + lax.axis_index("s")
    sec = wid // 4
    q = lax.rem(wid, 4)
    slot = lax.rem(sec, SPS)
    sec_row = sec * SEC
    tbl_base = sec * P2
    g0 = jnp.where(q == 0, 0, 6 * q + 1)
    n_g = jnp.where(q == 0, 7, 6)

    pltpu.sync_copy(idx_hbm.at[pl.ds(sec_row, SEC)], idx_v.at[pl.ds(0, SEC)])
    lanes = lax.iota(jnp.int32, 16)
    neg1 = jnp.full((16,), -1, jnp.int32)
    tail = idx_v[pl.ds(384, 16)]
    idx_v[pl.ds(384, 16)] = jnp.where(lanes < 8, tail, neg1)
    idx_v[pl.ds(400, 16)] = neg1
    idx_v[pl.ds(416, 16)] = neg1

    def stage(h, spm):
        @pl.when(q == 0)
        def _():
            pltpu.async_copy(
                tbl_hbm.at[pl.ds(tbl_base, 13), :, pl.ds(h * CW, CW)],
                spm.at[slot, pl.ds(0, 13)], gsem)

        @pl.when(q > 0)
        def _():
            start = 12 * q + 1
            pltpu.async_copy(
                tbl_hbm.at[pl.ds(tbl_base + start, 12), :, pl.ds(h * CW, CW)],
                spm.at[slot, pl.ds(start, 12)], gsem)

    def wait_stage(spm):
        @pl.when(q == 0)
        def _():
            pltpu.make_async_copy(
                tbl_hbm.at[pl.ds(0, 13), :, pl.ds(0, CW)],
                spm.at[slot, pl.ds(0, 13)], gsem).wait()

        @pl.when(q > 0)
        def _():
            pltpu.make_async_copy(
                tbl_hbm.at[pl.ds(0, 12), :, pl.ds(0, CW)],
                spm.at[slot, pl.ds(0, 12)], gsem).wait()

    def fan_out(h, spm):
        def fan_group(g, nw):
            grp = g0 + g
            v = idx_v[pl.ds(16 * grp, 16)]
            for j in range(16):
                t = v[j]
                p = grp * 16 + j

                @pl.when(t >= 0)
                def _():
                    pltpu.async_copy(
                        spm.at[slot, t],
                        out_hbm.at[sec_row + p, :, pl.ds(h * CW, CW)], wsem)

                nw = nw + jnp.where(t >= 0, 1, 0)
            return nw

        return lax.fori_loop(0, n_g, fan_group, jnp.int32(0))

    def drain(spm, count):
        def drain1(j, c):
            pltpu.make_async_copy(
                spm.at[slot, 0],
                out_hbm.at[0, :, pl.ds(0, CW)], wsem).wait()
            return c

        lax.fori_loop(0, count, drain1, 0)

    bufs = (spm_a, spm_b)
    stage(0, spm_a)
    for p in range(3):
        cur = bufs[p % 2]
        oth = bufs[1 - p % 2]
        wait_stage(cur)
        plsc.subcore_barrier()
        if p < 2:
            stage(p + 1, oth)
        nw = fan_out(p, cur)
        drain(cur, nw)


def kernel(r_idx, r_weight, kv):
    del r_weight
    idx = r_idx.reshape(ROWS).astype(jnp.int32)
    tbl = kv.reshape(N * P2, W2, CKV)
    out = _gather_kernel(idx, tbl)
    return out.reshape(N, P2, TOPK, W2, CKV)
